# baseline (speedup 1.0000x reference)
import jax
import jax.numpy as jnp
from jax import lax
from jax.experimental import pallas as pl
from jax.experimental.pallas import tpu as pltpu

_NB = 4


def _body(cam_s, idd_s, dt_s, img_ref, *refs):
    o_ref = refs[-1]
    tabs = refs[:-1]
    n0 = pl.program_id(0) * _NB
    for k in range(_NB):
        wc1, bc1, wi1, bi1, wc2, bc2, wi2, bi2 = tabs[8 * k:8 * k + 8]
        use1 = dt_s[n0 + k] == 0
        w = jnp.where(use1, wc1[0] + wi1[0], wc2[0] + wi2[0])
        b = jnp.where(use1, bc1[0] + bi1[0], bc2[0] + bi2[0])
        for c in range(3):
            wv = lax.slice(w, (0, c), (1, c + 1))
            bv = lax.slice(b, (0, c), (1, c + 1))
            o_ref[k, c] = img_ref[k, c] * wv + bv


def _row_spec(k, use_cam):
    if use_cam:
        return pl.BlockSpec(
            (1, 1, 3), lambda i, cam_s, idd_s, dt_s: (cam_s[_NB * i + k], 0, 0))
    return pl.BlockSpec(
        (1, 1, 3), lambda i, cam_s, idd_s, dt_s: (idd_s[_NB * i + k], 0, 0))


@jax.jit
def kernel(image, camindex, idindex, dataset_type,
           wcam1, bcam1, wident1, bident1,
           wcam2, bcam2, wident2, bident2):
    n, ch, h, wd = image.shape
    img_spec = pl.BlockSpec((_NB, ch, h, wd), lambda i, *_: (i, 0, 0, 0))
    tab_specs = []
    tab_args = []
    for k in range(_NB):
        for t, (tab, use_cam) in enumerate((
                (wcam1, True), (bcam1, True),
                (wident1, False), (bident1, False),
                (wcam2, True), (bcam2, True),
                (wident2, False), (bident2, False))):
            tab_specs.append(_row_spec(k, use_cam))
            tab_args.append(tab.reshape(-1, 1, 3))
    grid_spec = pltpu.PrefetchScalarGridSpec(
        num_scalar_prefetch=3,
        grid=(n // _NB,),
        in_specs=[img_spec] + tab_specs,
        out_specs=pl.BlockSpec((_NB, ch, h, wd), lambda i, *_: (i, 0, 0, 0)),
    )
    return pl.pallas_call(
        _body,
        grid_spec=grid_spec,
        out_shape=jax.ShapeDtypeStruct(image.shape, image.dtype),
        compiler_params=pltpu.CompilerParams(
            dimension_semantics=("arbitrary",)),
    )(camindex, idindex, dataset_type, image, *tab_args)


# speedup vs baseline: 1.0153x; 1.0153x over previous
import jax
import jax.numpy as jnp
from jax import lax
from jax.experimental import pallas as pl
from jax.experimental.pallas import tpu as pltpu

_NB = 4


def _body(cam_s, idd_s, dt_s, img_ref, *refs):
    o_ref = refs[-1]
    tabs = refs[:-1]
    n0 = pl.program_id(0) * _NB
    for k in range(_NB):
        wc1, bc1, wi1, bi1, wc2, bc2, wi2, bi2 = tabs[8 * k:8 * k + 8]
        use1 = dt_s[n0 + k] == 0
        w = jnp.where(use1, wc1[0] + wi1[0], wc2[0] + wi2[0])
        b = jnp.where(use1, bc1[0] + bi1[0], bc2[0] + bi2[0])
        for c in range(3):
            wv = lax.slice(w, (0, c), (1, c + 1))
            bv = lax.slice(b, (0, c), (1, c + 1))
            o_ref[k, c] = img_ref[k, c] * wv + bv


def _row_spec(k, use_cam):
    if use_cam:
        return pl.BlockSpec(
            (1, 1, 3), lambda i, cam_s, idd_s, dt_s: (cam_s[_NB * i + k], 0, 0))
    return pl.BlockSpec(
        (1, 1, 3), lambda i, cam_s, idd_s, dt_s: (idd_s[_NB * i + k], 0, 0))


@jax.jit
def kernel(image, camindex, idindex, dataset_type,
           wcam1, bcam1, wident1, bident1,
           wcam2, bcam2, wident2, bident2):
    n, ch, h, wd = image.shape
    img_spec = pl.BlockSpec((_NB, ch, h, wd), lambda i, *_: (i, 0, 0, 0))
    tab_specs = []
    tab_args = []
    for k in range(_NB):
        for t, (tab, use_cam) in enumerate((
                (wcam1, True), (bcam1, True),
                (wident1, False), (bident1, False),
                (wcam2, True), (bcam2, True),
                (wident2, False), (bident2, False))):
            tab_specs.append(_row_spec(k, use_cam))
            tab_args.append(tab.reshape(-1, 1, 3))
    grid_spec = pltpu.PrefetchScalarGridSpec(
        num_scalar_prefetch=3,
        grid=(n // _NB,),
        in_specs=[img_spec] + tab_specs,
        out_specs=pl.BlockSpec((_NB, ch, h, wd), lambda i, *_: (i, 0, 0, 0)),
    )
    return pl.pallas_call(
        _body,
        grid_spec=grid_spec,
        out_shape=jax.ShapeDtypeStruct(image.shape, image.dtype),
        compiler_params=pltpu.CompilerParams(
            dimension_semantics=("parallel",)),
    )(camindex, idindex, dataset_type, image, *tab_args)


# speedup vs baseline: 1.0215x; 1.0062x over previous
import jax
import jax.numpy as jnp
from jax import lax
from jax.experimental import pallas as pl
from jax.experimental.pallas import tpu as pltpu

_NB = 4


def _body(cam_s, idd_s, dt_s, img_ref, *refs):
    o_ref, wv_scr, bv_scr, ws_scr, bs_scr, sem = refs[-6:]
    tabs = refs[:-6]
    n0 = pl.program_id(0) * _NB
    for k in range(_NB):
        wc1, bc1, wi1, bi1, wc2, bc2, wi2, bi2 = tabs[8 * k:8 * k + 8]
        use1 = dt_s[n0 + k] == 0
        wv_scr[pl.ds(k, 1), :] = jnp.where(
            use1, wc1[0] + wi1[0], wc2[0] + wi2[0])
        bv_scr[pl.ds(k, 1), :] = jnp.where(
            use1, bc1[0] + bi1[0], bc2[0] + bi2[0])
    cw = pltpu.make_async_copy(wv_scr, ws_scr, sem)
    cw.start()
    cb = pltpu.make_async_copy(bv_scr, bs_scr, sem)
    cb.start()
    cw.wait()
    cb.wait()
    for k in range(_NB):
        for c in range(3):
            o_ref[k, c] = img_ref[k, c] * ws_scr[k, c] + bs_scr[k, c]


def _row_spec(k, use_cam):
    if use_cam:
        return pl.BlockSpec(
            (1, 1, 3), lambda i, cam_s, idd_s, dt_s: (cam_s[_NB * i + k], 0, 0))
    return pl.BlockSpec(
        (1, 1, 3), lambda i, cam_s, idd_s, dt_s: (idd_s[_NB * i + k], 0, 0))


@jax.jit
def kernel(image, camindex, idindex, dataset_type,
           wcam1, bcam1, wident1, bident1,
           wcam2, bcam2, wident2, bident2):
    n, ch, h, wd = image.shape
    img_spec = pl.BlockSpec((_NB, ch, h, wd), lambda i, *_: (i, 0, 0, 0))
    tab_specs = []
    tab_args = []
    for k in range(_NB):
        for t, (tab, use_cam) in enumerate((
                (wcam1, True), (bcam1, True),
                (wident1, False), (bident1, False),
                (wcam2, True), (bcam2, True),
                (wident2, False), (bident2, False))):
            tab_specs.append(_row_spec(k, use_cam))
            tab_args.append(tab.reshape(-1, 1, 3))
    grid_spec = pltpu.PrefetchScalarGridSpec(
        num_scalar_prefetch=3,
        grid=(n // _NB,),
        in_specs=[img_spec] + tab_specs,
        out_specs=pl.BlockSpec((_NB, ch, h, wd), lambda i, *_: (i, 0, 0, 0)),
        scratch_shapes=[
            pltpu.VMEM((_NB, 3), jnp.float32),
            pltpu.VMEM((_NB, 3), jnp.float32),
            pltpu.SMEM((_NB, 3), jnp.float32),
            pltpu.SMEM((_NB, 3), jnp.float32),
            pltpu.SemaphoreType.DMA,
        ],
    )
    return pl.pallas_call(
        _body,
        grid_spec=grid_spec,
        out_shape=jax.ShapeDtypeStruct(image.shape, image.dtype),
        compiler_params=pltpu.CompilerParams(
            dimension_semantics=("parallel",)),
    )(camindex, idindex, dataset_type, image, *tab_args)
